# baseline (speedup 1.0000x reference)
import jax
import jax.numpy as jnp
from jax.experimental import pallas as pl


_BLK = 2048


def _broadcast_body(a_ref, o_ref):
    o_ref[...] = jnp.broadcast_to(a_ref[...], o_ref.shape)


def kernel(x, action):
    B = x.shape[0]
    A = action.shape[0]
    a2 = action.reshape(1, A)
    grid = (B // _BLK,)
    return pl.pallas_call(
        _broadcast_body,
        grid=grid,
        in_specs=[pl.BlockSpec((1, A), lambda i: (0, 0))],
        out_specs=pl.BlockSpec((_BLK, A), lambda i: (i, 0)),
        out_shape=jax.ShapeDtypeStruct((B, A), jnp.float32),
    )(a2)


# speedup vs baseline: 1.0916x; 1.0916x over previous
import jax
import jax.numpy as jnp
from jax.experimental import pallas as pl


_BLK = 2048


def _broadcast_body(a_ref, o_ref):
    o_ref[...] = jnp.broadcast_to(a_ref[...], o_ref.shape)


def kernel(x, action):
    B = x.shape[0]
    A = action.shape[0]
    a2 = action.reshape(1, A)
    return pl.pallas_call(
        _broadcast_body,
        out_shape=jax.ShapeDtypeStruct((B, A), jnp.float32),
    )(a2)


# speedup vs baseline: 1.1347x; 1.0395x over previous
import jax
import jax.numpy as jnp
from jax.experimental import pallas as pl
from jax.experimental.pallas import tpu as pltpu

_K = 8


def _body(a_ref, o_hbm, buf, sems):
    buf[...] = jnp.broadcast_to(a_ref[...], buf.shape)
    blk = buf.shape[0]
    copies = [
        pltpu.make_async_copy(buf, o_hbm.at[pl.ds(k * blk, blk), :], sems.at[k])
        for k in range(_K)
    ]
    for c in copies:
        c.start()
    for c in copies:
        c.wait()


def kernel(x, action):
    B = x.shape[0]
    A = action.shape[0]
    a2 = action.reshape(1, A)
    blk = B // _K
    return pl.pallas_call(
        _body,
        in_specs=[pl.BlockSpec((1, A), lambda: (0, 0))],
        out_specs=pl.BlockSpec(memory_space=pl.ANY),
        out_shape=jax.ShapeDtypeStruct((B, A), jnp.float32),
        scratch_shapes=[
            pltpu.VMEM((blk, A), jnp.float32),
            pltpu.SemaphoreType.DMA((_K,)),
        ],
    )(a2)
